# baseline (speedup 1.0000x reference)
import functools

import jax
import jax.numpy as jnp
from jax import lax
from jax.experimental import pallas as pl
from jax.experimental.pallas import tpu as pltpu
from jax.experimental.pallas import tpu_sc as plsc

N = 10000
E = 320000
NUM_HE = 10000
NUM_GRAPHS = 64
ATOM_FEA_DIM = 128
F = 64
H_DIM = 128
NUM_LAYERS = 3

_NC = 2
_NS = 16
_NW = _NC * _NS
_LANES = 16
_W = 128
_C = 128
_NCHUNK = 80
_EPT = _NCHUNK * _C
_HC = _NCHUNK // 2
_NP = 10240
_RPT = _NP // _NS
_ZC = 128

_HIGH = jax.lax.Precision.DEFAULT


def _dot(a, b):
    return lax.dot_general(a, b, (((1,), (0,)), ((), ())),
                           precision=_HIGH, preferred_element_type=jnp.float32)


@functools.partial(
    pl.kernel,
    mesh=plsc.VectorSubcoreMesh(core_axis_name="c", subcore_axis_name="s"),
    out_type=jax.ShapeDtypeStruct((_NC, _NP, _W), jnp.float32),
    scratch_types=[
        pltpu.VMEM((_HC, _C), jnp.int32),
        pltpu.VMEM((_HC, _C), jnp.int32),
        pltpu.VMEM((2, _C, _W), jnp.float32),
        pltpu.VMEM_SHARED((_NP, _W), jnp.float32),
        pltpu.SemaphoreType.DMA,
        pltpu.SemaphoreType.DMA,
        pltpu.SemaphoreType.DMA,
        pltpu.SemaphoreType.DMA,
    ],
)
def _sc_pass(src_hbm, sidx_hbm, didx_hbm, out_hbm,
             sidx_v, didx_v, rows_v, accum_s, gsem0, gsem1, ssem0, ssem1):
    cid = lax.axis_index("c")
    sid = lax.axis_index("s")
    wid = sid * _NC + cid

    def _zrow(r, carry):
        for j in range(_W // _LANES):
            rows_v[0, r, pl.ds(j * _LANES, _LANES)] = jnp.zeros((_LANES,), jnp.float32)
        return carry
    lax.fori_loop(0, _ZC, _zrow, 0)
    base = sid * _RPT
    for z in range(_RPT // _ZC):
        pltpu.sync_copy(rows_v.at[0], accum_s.at[pl.ds(base + z * _ZC, _ZC)])
    plsc.subcore_barrier()

    def _gather(c, buf, sem):
        return pltpu.async_copy(src_hbm.at[sidx_v.at[c]], rows_v.at[buf], sem)

    def _gwait(c, buf, sem):
        pltpu.make_async_copy(src_hbm.at[sidx_v.at[c]], rows_v.at[buf],
                              sem).wait()

    def _scatter(c, buf, sem):
        return pltpu.async_copy(rows_v.at[buf], accum_s.at[didx_v.at[c]], sem,
                                add=True)

    def _swait(c, buf, sem):
        pltpu.make_async_copy(rows_v.at[buf], accum_s.at[didx_v.at[c]],
                              sem).wait()

    for h in range(2):
        pltpu.sync_copy(sidx_hbm.at[wid, pl.ds(h * _HC, _HC)], sidx_v)
        pltpu.sync_copy(didx_hbm.at[wid, pl.ds(h * _HC, _HC)], didx_v)
        _gather(0, 0, gsem0)
        _gather(1, 1, gsem1)
        _gwait(0, 0, gsem0)
        _scatter(0, 0, ssem0)

        def _body(k, carry):
            c0 = 2 * k
            _gwait(c0 + 1, 1, gsem1)
            _scatter(c0 + 1, 1, ssem1)
            _swait(c0, 0, ssem0)
            _gather(c0 + 2, 0, gsem0)
            _gwait(c0 + 2, 0, gsem0)
            _scatter(c0 + 2, 0, ssem0)
            _swait(c0 + 1, 1, ssem1)
            _gather(c0 + 3, 1, gsem1)
            return carry
        lax.fori_loop(0, _HC // 2 - 1, _body, 0)

        c0 = _HC - 2
        _gwait(c0 + 1, 1, gsem1)
        _scatter(c0 + 1, 1, ssem1)
        _swait(c0, 0, ssem0)
        _swait(c0 + 1, 1, ssem1)
    plsc.subcore_barrier()

    pltpu.sync_copy(accum_s.at[pl.ds(base, _RPT)],
                    out_hbm.at[cid, pl.ds(base, _RPT)])


def _pack(o_ref, vals):
    o_ref[:, 0:F] = vals
    o_ref[:, F:F + 1] = jnp.ones((vals.shape[0], 1), jnp.float32)
    o_ref[:, F + 1:_W] = jnp.zeros((vals.shape[0], _W - F - 1), jnp.float32)


def _tc_embed_body(x_ref, w_ref, b_ref, t_ref, o_ref):
    h = _dot(x_ref[...], w_ref[...]) + b_ref[...]
    _pack(o_ref, _dot(h, t_ref[...]))


_tc_embed = pl.pallas_call(
    _tc_embed_body, out_shape=jax.ShapeDtypeStruct((N, _W), jnp.float32))


def _seg_inv(p_ref):
    s = p_ref[0, 0:N, 0:F] + p_ref[1, 0:N, 0:F]
    cnt = p_ref[0, 0:N, F:F + 1] + p_ref[1, 0:N, F:F + 1]
    inv = jnp.where(cnt > 0, 1.0 / jnp.maximum(cnt, 1.0), 0.0)
    return s, inv


def _tc_mid_body(p_ref, o_ref):
    s, binv = _seg_inv(p_ref)
    _pack(o_ref, s * binv)


_tc_mid = pl.pallas_call(
    _tc_mid_body, out_shape=jax.ShapeDtypeStruct((NUM_HE, _W), jnp.float32))


def _bn(o, gamma, beta):
    mean = jnp.mean(o, axis=0, keepdims=True)
    var = jnp.mean((o - mean) ** 2, axis=0, keepdims=True)
    return gamma * (o - mean) / jnp.sqrt(var + 1e-5) + beta


def _tc_post_body(p_ref, bias_ref, gamma_ref, beta_ref, t_ref, o_ref):
    s, dinv = _seg_inv(p_ref)
    o = s * dinv + bias_ref[...]
    h = _bn(o, gamma_ref[...], beta_ref[...])
    _pack(o_ref, _dot(h, t_ref[...]))


_tc_post = pl.pallas_call(
    _tc_post_body, out_shape=jax.ShapeDtypeStruct((N, _W), jnp.float32))


def _softplus(x):
    m = jnp.maximum(x, 0.0)
    return m + jnp.log(jnp.exp(x - m) + jnp.exp(-m))


def _tc_final_body(p_ref, bias_ref, gamma_ref, beta_ref, batch_ref,
                   fcw_ref, fcb_ref, ow_ref, ob_ref, o_ref):
    s, dinv = _seg_inv(p_ref)
    o = s * dinv + bias_ref[...]
    h = _bn(o, gamma_ref[...], beta_ref[...])
    gids = lax.broadcasted_iota(jnp.int32, (1, NUM_GRAPHS), 1)
    onehot = (batch_ref[...] == gids).astype(jnp.float32)
    sums = lax.dot_general(onehot, h, (((0,), (0,)), ((), ())),
                           precision=_HIGH, preferred_element_type=jnp.float32)
    counts = lax.dot_general(onehot, jnp.ones((N, 1), jnp.float32),
                             (((0,), (0,)), ((), ())),
                             precision=_HIGH, preferred_element_type=jnp.float32)
    pooled = sums / jnp.maximum(counts, 1.0)
    p = _softplus(pooled)
    p = _softplus(_dot(p, fcw_ref[...]) + fcb_ref[...])
    o_ref[...] = _dot(p, ow_ref[...]) + ob_ref[...]


_tc_final = pl.pallas_call(
    _tc_final_body, out_shape=jax.ShapeDtypeStruct((NUM_GRAPHS, 1), jnp.float32))


def _prep_idx(idx, scatter_side):
    per = E // _NW
    npad = _EPT - per
    t = idx.reshape(_NW, per)
    j = jnp.arange(_NW * npad, dtype=idx.dtype).reshape(_NW, npad)
    pad = N + (j % (_NP - N)) if scatter_side else (j * 89) % N
    return jnp.concatenate([t, pad.astype(idx.dtype)], axis=1).reshape(
        _NW, _NCHUNK, _C)


def kernel(x, hyperedge_index, batch, emb_W, emb_b, thetas, conv_bias,
           gammas, betas, fc_W, fc_b, out_W, out_b):
    node_s = _prep_idx(hyperedge_index[0], False)
    node_d = _prep_idx(hyperedge_index[0], True)
    he_s = _prep_idx(hyperedge_index[1], False)
    he_d = _prep_idx(hyperedge_index[1], True)

    g = _tc_embed(x, emb_W, emb_b.reshape(1, F), thetas[0])
    for l in range(NUM_LAYERS):
        p1 = _sc_pass(g, node_s, he_d)
        ef = _tc_mid(p1)
        p2 = _sc_pass(ef, he_s, node_d)
        if l < NUM_LAYERS - 1:
            g = _tc_post(p2, conv_bias[l].reshape(1, F),
                         gammas[l].reshape(1, F), betas[l].reshape(1, F),
                         thetas[l + 1])
        else:
            out = _tc_final(p2, conv_bias[l].reshape(1, F),
                            gammas[l].reshape(1, F), betas[l].reshape(1, F),
                            batch.reshape(N, 1), fc_W, fc_b.reshape(1, H_DIM),
                            out_W, out_b.reshape(1, 1))
    return out


# speedup vs baseline: 1.1754x; 1.1754x over previous
import functools

import jax
import jax.numpy as jnp
from jax import lax
from jax.experimental import pallas as pl
from jax.experimental.pallas import tpu as pltpu
from jax.experimental.pallas import tpu_sc as plsc

N = 10000
E = 320000
NUM_HE = 10000
NUM_GRAPHS = 64
ATOM_FEA_DIM = 128
F = 64
H_DIM = 128
NUM_LAYERS = 3

_NC = 2
_NS = 16
_NW = _NC * _NS
_LANES = 16
_W = 128
_C = 128
_NCHUNK = 80
_EPT = _NCHUNK * _C
_HC = _NCHUNK // 2
_NP = 10240
_RPT = _NP // _NS
_ZC = 128

_HIGH = jax.lax.Precision.DEFAULT


def _dot(a, b):
    return lax.dot_general(a, b, (((1,), (0,)), ((), ())),
                           precision=_HIGH, preferred_element_type=jnp.float32)


@functools.partial(
    pl.kernel,
    mesh=plsc.VectorSubcoreMesh(core_axis_name="c", subcore_axis_name="s"),
    out_type=jax.ShapeDtypeStruct((_NC, _NP, _W), jnp.float32),
    scratch_types=[
        pltpu.VMEM((_HC, _C), jnp.int32),
        pltpu.VMEM((_HC, _C), jnp.int32),
        pltpu.VMEM((2, _C, _W), jnp.float32),
        pltpu.VMEM_SHARED((_NP, _W), jnp.float32),
        pltpu.SemaphoreType.DMA,
        pltpu.SemaphoreType.DMA,
    ],
)
def _sc_pass(src_hbm, sidx_hbm, didx_hbm, out_hbm,
             sidx_v, didx_v, rows_v, accum_s, gsem0, gsem1):
    cid = lax.axis_index("c")
    sid = lax.axis_index("s")
    wid = sid * _NC + cid

    def _zrow(r, carry):
        for j in range(_W // _LANES):
            rows_v[0, r, pl.ds(j * _LANES, _LANES)] = jnp.zeros((_LANES,), jnp.float32)
        return carry
    lax.fori_loop(0, _ZC, _zrow, 0)
    base = sid * _RPT
    for z in range(_RPT // _ZC):
        pltpu.sync_copy(rows_v.at[0], accum_s.at[pl.ds(base + z * _ZC, _ZC)])
    plsc.subcore_barrier()

    def _gather(c, buf, sem):
        return pltpu.async_copy(src_hbm.at[sidx_v.at[c]], rows_v.at[buf], sem)

    def _gwait(c, buf, sem):
        pltpu.make_async_copy(src_hbm.at[sidx_v.at[c]], rows_v.at[buf],
                              sem).wait()

    def _scatter(c, buf):
        pltpu.sync_copy(rows_v.at[buf], accum_s.at[didx_v.at[c]], add=True)

    for h in range(2):
        pltpu.sync_copy(sidx_hbm.at[wid, pl.ds(h * _HC, _HC)], sidx_v)
        pltpu.sync_copy(didx_hbm.at[wid, pl.ds(h * _HC, _HC)], didx_v)
        _gather(0, 0, gsem0)

        def _body(k, carry):
            c0 = 2 * k
            _gather(c0 + 1, 1, gsem1)
            _gwait(c0, 0, gsem0)
            _scatter(c0, 0)
            _gather(c0 + 2, 0, gsem0)
            _gwait(c0 + 1, 1, gsem1)
            _scatter(c0 + 1, 1)
            return carry
        lax.fori_loop(0, _HC // 2 - 1, _body, 0)

        c0 = _HC - 2
        _gather(c0 + 1, 1, gsem1)
        _gwait(c0, 0, gsem0)
        _scatter(c0, 0)
        _gwait(c0 + 1, 1, gsem1)
        _scatter(c0 + 1, 1)
    plsc.subcore_barrier()

    pltpu.sync_copy(accum_s.at[pl.ds(base, _RPT)],
                    out_hbm.at[cid, pl.ds(base, _RPT)])


def _pack(o_ref, vals):
    o_ref[:, 0:F] = vals
    o_ref[:, F:F + 1] = jnp.ones((vals.shape[0], 1), jnp.float32)
    o_ref[:, F + 1:_W] = jnp.zeros((vals.shape[0], _W - F - 1), jnp.float32)


def _tc_embed_body(x_ref, w_ref, b_ref, t_ref, o_ref):
    h = _dot(x_ref[...], w_ref[...]) + b_ref[...]
    _pack(o_ref, _dot(h, t_ref[...]))


_tc_embed = pl.pallas_call(
    _tc_embed_body, out_shape=jax.ShapeDtypeStruct((N, _W), jnp.float32))


def _seg_inv(p_ref):
    s = p_ref[0, 0:N, 0:F] + p_ref[1, 0:N, 0:F]
    cnt = p_ref[0, 0:N, F:F + 1] + p_ref[1, 0:N, F:F + 1]
    inv = jnp.where(cnt > 0, 1.0 / jnp.maximum(cnt, 1.0), 0.0)
    return s, inv


def _tc_mid_body(p_ref, o_ref):
    s, binv = _seg_inv(p_ref)
    _pack(o_ref, s * binv)


_tc_mid = pl.pallas_call(
    _tc_mid_body, out_shape=jax.ShapeDtypeStruct((NUM_HE, _W), jnp.float32))


def _bn(o, gamma, beta):
    mean = jnp.mean(o, axis=0, keepdims=True)
    var = jnp.mean((o - mean) ** 2, axis=0, keepdims=True)
    return gamma * (o - mean) / jnp.sqrt(var + 1e-5) + beta


def _tc_post_body(p_ref, bias_ref, gamma_ref, beta_ref, t_ref, o_ref):
    s, dinv = _seg_inv(p_ref)
    o = s * dinv + bias_ref[...]
    h = _bn(o, gamma_ref[...], beta_ref[...])
    _pack(o_ref, _dot(h, t_ref[...]))


_tc_post = pl.pallas_call(
    _tc_post_body, out_shape=jax.ShapeDtypeStruct((N, _W), jnp.float32))


def _softplus(x):
    m = jnp.maximum(x, 0.0)
    return m + jnp.log(jnp.exp(x - m) + jnp.exp(-m))


def _tc_final_body(p_ref, bias_ref, gamma_ref, beta_ref, batch_ref,
                   fcw_ref, fcb_ref, ow_ref, ob_ref, o_ref):
    s, dinv = _seg_inv(p_ref)
    o = s * dinv + bias_ref[...]
    h = _bn(o, gamma_ref[...], beta_ref[...])
    gids = lax.broadcasted_iota(jnp.int32, (1, NUM_GRAPHS), 1)
    onehot = (batch_ref[...] == gids).astype(jnp.float32)
    sums = lax.dot_general(onehot, h, (((0,), (0,)), ((), ())),
                           precision=_HIGH, preferred_element_type=jnp.float32)
    counts = lax.dot_general(onehot, jnp.ones((N, 1), jnp.float32),
                             (((0,), (0,)), ((), ())),
                             precision=_HIGH, preferred_element_type=jnp.float32)
    pooled = sums / jnp.maximum(counts, 1.0)
    p = _softplus(pooled)
    p = _softplus(_dot(p, fcw_ref[...]) + fcb_ref[...])
    o_ref[...] = _dot(p, ow_ref[...]) + ob_ref[...]


_tc_final = pl.pallas_call(
    _tc_final_body, out_shape=jax.ShapeDtypeStruct((NUM_GRAPHS, 1), jnp.float32))


def _prep_idx(idx, scatter_side):
    per = E // _NW
    npad = _EPT - per
    t = idx.reshape(_NW, per)
    j = jnp.arange(_NW * npad, dtype=idx.dtype).reshape(_NW, npad)
    pad = N + (j % (_NP - N)) if scatter_side else (j * 89) % N
    return jnp.concatenate([t, pad.astype(idx.dtype)], axis=1).reshape(
        _NW, _NCHUNK, _C)


def kernel(x, hyperedge_index, batch, emb_W, emb_b, thetas, conv_bias,
           gammas, betas, fc_W, fc_b, out_W, out_b):
    node_s = _prep_idx(hyperedge_index[0], False)
    node_d = _prep_idx(hyperedge_index[0], True)
    he_s = _prep_idx(hyperedge_index[1], False)
    he_d = _prep_idx(hyperedge_index[1], True)

    g = _tc_embed(x, emb_W, emb_b.reshape(1, F), thetas[0])
    for l in range(NUM_LAYERS):
        p1 = _sc_pass(g, node_s, he_d)
        ef = _tc_mid(p1)
        p2 = _sc_pass(ef, he_s, node_d)
        if l < NUM_LAYERS - 1:
            g = _tc_post(p2, conv_bias[l].reshape(1, F),
                         gammas[l].reshape(1, F), betas[l].reshape(1, F),
                         thetas[l + 1])
        else:
            out = _tc_final(p2, conv_bias[l].reshape(1, F),
                            gammas[l].reshape(1, F), betas[l].reshape(1, F),
                            batch.reshape(N, 1), fc_W, fc_b.reshape(1, H_DIM),
                            out_W, out_b.reshape(1, 1))
    return out


# speedup vs baseline: 1.1945x; 1.0163x over previous
import functools

import jax
import jax.numpy as jnp
from jax import lax
from jax.experimental import pallas as pl
from jax.experimental.pallas import tpu as pltpu
from jax.experimental.pallas import tpu_sc as plsc

N = 10000
E = 320000
NUM_HE = 10000
NUM_GRAPHS = 64
ATOM_FEA_DIM = 128
F = 64
H_DIM = 128
NUM_LAYERS = 3

_NC = 2
_NS = 16
_NW = _NC * _NS
_LANES = 16
_W = 128
_C = 128
_NCHUNK = 80
_EPT = _NCHUNK * _C
_HC = _NCHUNK // 2
_NP = 10240
_RPT = _NP // _NS
_ZC = 128

_HIGH = jax.lax.Precision.DEFAULT


def _dot(a, b):
    return lax.dot_general(a, b, (((1,), (0,)), ((), ())),
                           precision=_HIGH, preferred_element_type=jnp.float32)


@functools.partial(
    pl.kernel,
    mesh=plsc.VectorSubcoreMesh(core_axis_name="c", subcore_axis_name="s"),
    out_type=jax.ShapeDtypeStruct((_NC, _NP, _W), jnp.float32),
    scratch_types=[
        pltpu.VMEM((_HC, _C), jnp.int32),
        pltpu.VMEM((_HC, _C), jnp.int32),
        pltpu.VMEM((2, _C, _W), jnp.float32),
        pltpu.VMEM_SHARED((_NP, _W), jnp.float32),
        pltpu.SemaphoreType.DMA,
        pltpu.SemaphoreType.DMA,
        pltpu.SemaphoreType.DMA,
    ],
)
def _sc_pass(src_hbm, sidx_hbm, didx_hbm, out_hbm,
             sidx_v, didx_v, rows_v, accum_s, gsem0, gsem1, zsem):
    cid = lax.axis_index("c")
    sid = lax.axis_index("s")
    wid = sid * _NC + cid

    s_stage = pltpu.async_copy(sidx_hbm.at[wid, pl.ds(0, _HC)], sidx_v, gsem0)
    d_stage = pltpu.async_copy(didx_hbm.at[wid, pl.ds(0, _HC)], didx_v, gsem1)

    def _zrow(r, carry):
        for j in range(_W // _LANES):
            rows_v[0, r, pl.ds(j * _LANES, _LANES)] = jnp.zeros((_LANES,), jnp.float32)
        return carry
    lax.fori_loop(0, _ZC, _zrow, 0)
    base = sid * _RPT
    zcps = [pltpu.async_copy(rows_v.at[0],
                             accum_s.at[pl.ds(base + z * _ZC, _ZC)], zsem)
            for z in range(_RPT // _ZC)]
    s_stage.wait()
    d_stage.wait()
    for cp in zcps:
        cp.wait()
    plsc.subcore_barrier()

    def _gather(c, buf, sem):
        return pltpu.async_copy(src_hbm.at[sidx_v.at[c]], rows_v.at[buf], sem)

    def _gwait(c, buf, sem):
        pltpu.make_async_copy(src_hbm.at[sidx_v.at[c]], rows_v.at[buf],
                              sem).wait()

    def _scatter(c, buf):
        pltpu.sync_copy(rows_v.at[buf], accum_s.at[didx_v.at[c]], add=True)

    for h in range(2):
        if h:
            pltpu.sync_copy(sidx_hbm.at[wid, pl.ds(h * _HC, _HC)], sidx_v)
            pltpu.sync_copy(didx_hbm.at[wid, pl.ds(h * _HC, _HC)], didx_v)
        _gather(0, 0, gsem0)

        def _body(k, carry):
            c0 = 2 * k
            _gather(c0 + 1, 1, gsem1)
            _gwait(c0, 0, gsem0)
            _scatter(c0, 0)
            _gather(c0 + 2, 0, gsem0)
            _gwait(c0 + 1, 1, gsem1)
            _scatter(c0 + 1, 1)
            return carry
        lax.fori_loop(0, _HC // 2 - 1, _body, 0)

        c0 = _HC - 2
        _gather(c0 + 1, 1, gsem1)
        _gwait(c0, 0, gsem0)
        _scatter(c0, 0)
        _gwait(c0 + 1, 1, gsem1)
        _scatter(c0 + 1, 1)
    plsc.subcore_barrier()

    pltpu.sync_copy(accum_s.at[pl.ds(base, _RPT)],
                    out_hbm.at[cid, pl.ds(base, _RPT)])


def _pack(o_ref, vals):
    o_ref[:, 0:F] = vals
    o_ref[:, F:F + 1] = jnp.ones((vals.shape[0], 1), jnp.float32)
    o_ref[:, F + 1:_W] = jnp.zeros((vals.shape[0], _W - F - 1), jnp.float32)


def _tc_embed_body(x_ref, w_ref, b_ref, t_ref, o_ref):
    h = _dot(x_ref[...], w_ref[...]) + b_ref[...]
    _pack(o_ref, _dot(h, t_ref[...]))


_tc_embed = pl.pallas_call(
    _tc_embed_body, out_shape=jax.ShapeDtypeStruct((N, _W), jnp.float32))


def _seg_inv(p_ref):
    s = p_ref[0, 0:N, 0:F] + p_ref[1, 0:N, 0:F]
    cnt = p_ref[0, 0:N, F:F + 1] + p_ref[1, 0:N, F:F + 1]
    inv = jnp.where(cnt > 0, 1.0 / jnp.maximum(cnt, 1.0), 0.0)
    return s, inv


def _tc_mid_body(p_ref, o_ref):
    s, binv = _seg_inv(p_ref)
    _pack(o_ref, s * binv)


_tc_mid = pl.pallas_call(
    _tc_mid_body, out_shape=jax.ShapeDtypeStruct((NUM_HE, _W), jnp.float32))


def _bn(o, gamma, beta):
    mean = jnp.mean(o, axis=0, keepdims=True)
    var = jnp.mean((o - mean) ** 2, axis=0, keepdims=True)
    return gamma * (o - mean) / jnp.sqrt(var + 1e-5) + beta


def _tc_post_body(p_ref, bias_ref, gamma_ref, beta_ref, t_ref, o_ref):
    s, dinv = _seg_inv(p_ref)
    o = s * dinv + bias_ref[...]
    h = _bn(o, gamma_ref[...], beta_ref[...])
    _pack(o_ref, _dot(h, t_ref[...]))


_tc_post = pl.pallas_call(
    _tc_post_body, out_shape=jax.ShapeDtypeStruct((N, _W), jnp.float32))


def _softplus(x):
    m = jnp.maximum(x, 0.0)
    return m + jnp.log(jnp.exp(x - m) + jnp.exp(-m))


def _tc_final_body(p_ref, bias_ref, gamma_ref, beta_ref, batch_ref,
                   fcw_ref, fcb_ref, ow_ref, ob_ref, o_ref):
    s, dinv = _seg_inv(p_ref)
    o = s * dinv + bias_ref[...]
    h = _bn(o, gamma_ref[...], beta_ref[...])
    gids = lax.broadcasted_iota(jnp.int32, (1, NUM_GRAPHS), 1)
    onehot = (batch_ref[...] == gids).astype(jnp.float32)
    sums = lax.dot_general(onehot, h, (((0,), (0,)), ((), ())),
                           precision=_HIGH, preferred_element_type=jnp.float32)
    counts = lax.dot_general(onehot, jnp.ones((N, 1), jnp.float32),
                             (((0,), (0,)), ((), ())),
                             precision=_HIGH, preferred_element_type=jnp.float32)
    pooled = sums / jnp.maximum(counts, 1.0)
    p = _softplus(pooled)
    p = _softplus(_dot(p, fcw_ref[...]) + fcb_ref[...])
    o_ref[...] = _dot(p, ow_ref[...]) + ob_ref[...]


_tc_final = pl.pallas_call(
    _tc_final_body, out_shape=jax.ShapeDtypeStruct((NUM_GRAPHS, 1), jnp.float32))


def _prep_idx(idx, scatter_side):
    per = E // _NW
    npad = _EPT - per
    t = idx.reshape(_NW, per)
    j = jnp.arange(_NW * npad, dtype=idx.dtype).reshape(_NW, npad)
    pad = N + (j % (_NP - N)) if scatter_side else (j * 89) % N
    return jnp.concatenate([t, pad.astype(idx.dtype)], axis=1).reshape(
        _NW, _NCHUNK, _C)


def kernel(x, hyperedge_index, batch, emb_W, emb_b, thetas, conv_bias,
           gammas, betas, fc_W, fc_b, out_W, out_b):
    node_s = _prep_idx(hyperedge_index[0], False)
    node_d = _prep_idx(hyperedge_index[0], True)
    he_s = _prep_idx(hyperedge_index[1], False)
    he_d = _prep_idx(hyperedge_index[1], True)

    g = _tc_embed(x, emb_W, emb_b.reshape(1, F), thetas[0])
    for l in range(NUM_LAYERS):
        p1 = _sc_pass(g, node_s, he_d)
        ef = _tc_mid(p1)
        p2 = _sc_pass(ef, he_s, node_d)
        if l < NUM_LAYERS - 1:
            g = _tc_post(p2, conv_bias[l].reshape(1, F),
                         gammas[l].reshape(1, F), betas[l].reshape(1, F),
                         thetas[l + 1])
        else:
            out = _tc_final(p2, conv_bias[l].reshape(1, F),
                            gammas[l].reshape(1, F), betas[l].reshape(1, F),
                            batch.reshape(N, 1), fc_W, fc_b.reshape(1, H_DIM),
                            out_W, out_b.reshape(1, 1))
    return out
